# Initial kernel scaffold; baseline (speedup 1.0000x reference)
import jax
import jax.numpy as jnp
from jax.experimental import pallas as pl


def kernel(x, w, b):
    raise NotImplementedError("write your pallas kernel here")


# speedup vs baseline: 2.3760x; 2.3760x over previous
import functools

import jax
import jax.numpy as jnp
from jax import lax
from jax.experimental import pallas as pl
from jax.experimental.pallas import tpu as pltpu
from jax.experimental.pallas import tpu_sc as plsc

B = 128
N = 32768
NH = N // 2
NC = 2
NS = 16
NW = NC * NS

CSTR = 8
W = N // CSTR
W2 = W // 2
RGRP = NW // CSTR
RPW = B // RGRP
L = 16


def _body(x_hbm, w_hbm, b_hbm, out_hbm, xb, ob, wt, bt):
    wid = lax.axis_index("s") * NC + lax.axis_index("c")
    cs = wid % CSTR
    rg = wid // CSTR
    c0 = pl.multiple_of(cs * W, 8)
    h0 = pl.multiple_of(cs * W2, 8)
    r0 = rg * RPW

    pltpu.sync_copy(w_hbm.at[pl.ds(h0, W2)], wt)
    pltpu.sync_copy(b_hbm.at[pl.ds(h0, W2)], bt)

    nanv = jnp.full((L,), jnp.nan, dtype=jnp.float32)

    @pl.loop(0, W, step=L)
    def _fill(o):
        ob[pl.ds(o, L)] = nanv

    iota2 = lax.iota(jnp.int32, L) * 2

    @pl.loop(0, RPW)
    def _row(r):
        row = r0 + r
        pltpu.sync_copy(x_hbm.at[row, pl.ds(c0, W)], xb)

        @pl.loop(0, W2, step=L)
        def _vec(j):
            ie = j * 2 + iota2
            xe = plsc.load_gather(xb, [ie])
            p = xe * wt[pl.ds(j, L)] + bt[pl.ds(j, L)]
            plsc.store_scatter(ob, [ie + 1], p)

        pltpu.sync_copy(ob, out_hbm.at[row, 0, pl.ds(c0, W)])


def kernel(x, w, b):
    mesh = plsc.VectorSubcoreMesh(core_axis_name="c", subcore_axis_name="s")
    run = functools.partial(
        pl.kernel,
        out_type=jax.ShapeDtypeStruct((B, 1, N), jnp.float32),
        mesh=mesh,
        scratch_types=[
            pltpu.VMEM((W,), jnp.float32),
            pltpu.VMEM((W,), jnp.float32),
            pltpu.VMEM((W2,), jnp.float32),
            pltpu.VMEM((W2,), jnp.float32),
        ],
        compiler_params=pltpu.CompilerParams(needs_layout_passes=False),
    )(_body)
    return run(x, w, b)


# speedup vs baseline: 3.8963x; 1.6398x over previous
import functools

import jax
import jax.numpy as jnp
from jax import lax
from jax.experimental import pallas as pl
from jax.experimental.pallas import tpu as pltpu
from jax.experimental.pallas import tpu_sc as plsc

B = 128
N = 32768
NH = N // 2
NC = 2
NS = 16
NW = NC * NS

CSTR = 8
W = N // CSTR
W2 = W // 2
RGRP = NW // CSTR
RPW = B // RGRP
RB = 4
NB = RPW // RB
NBP = NB // 2
L = 16


def _body(x_hbm, w_hbm, b_hbm, out_hbm,
          xb0, xb1, ob0, ob1, wt, bt, si0, si1, so0, so1):
    wid = lax.axis_index("s") * NC + lax.axis_index("c")
    cs = wid % CSTR
    rg = wid // CSTR
    c0 = pl.multiple_of(cs * W, 8)
    h0 = pl.multiple_of(cs * W2, 8)
    r0 = rg * RPW

    pltpu.sync_copy(w_hbm.at[pl.ds(h0, W2)], wt)
    pltpu.sync_copy(b_hbm.at[pl.ds(h0, W2)], bt)

    nanv = jnp.full((L,), jnp.nan, dtype=jnp.float32)

    @pl.loop(0, W, step=L)
    def _fill(o):
        for r in range(RB):
            ob0[r, pl.ds(o, L)] = nanv
            ob1[r, pl.ds(o, L)] = nanv

    iota2 = lax.iota(jnp.int32, L) * 2

    def in_copy(blk, xb, sem):
        row = r0 + blk * RB
        return pltpu.make_async_copy(
            x_hbm.at[pl.ds(row, RB), pl.ds(c0, W)], xb, sem)

    def out_copy(blk, ob, sem):
        row = r0 + blk * RB
        return pltpu.make_async_copy(
            ob, out_hbm.at[pl.ds(row, RB), 0, pl.ds(c0, W)], sem)

    def compute(xb, ob):
        @pl.loop(0, W2, step=L)
        def _vec(j):
            ie = j * 2 + iota2
            io = ie + 1
            wv = wt[pl.ds(j, L)]
            bv = bt[pl.ds(j, L)]
            for r in range(RB):
                rv = jnp.full((L,), r, dtype=jnp.int32)
                xe = plsc.load_gather(xb, [rv, ie])
                plsc.store_scatter(ob, [rv, io], xe * wv + bv)

    in_copy(0, xb0, si0).start()
    in_copy(1, xb1, si1).start()

    @pl.loop(0, NBP)
    def _pair(g):
        blk0 = g * 2
        blk1 = blk0 + 1

        in_copy(blk0, xb0, si0).wait()

        @pl.when(g > 0)
        def _():
            out_copy(blk0 - 2, ob0, so0).wait()

        compute(xb0, ob0)
        out_copy(blk0, ob0, so0).start()

        @pl.when(g < NBP - 1)
        def _():
            in_copy(blk0 + 2, xb0, si0).start()

        in_copy(blk1, xb1, si1).wait()

        @pl.when(g > 0)
        def _():
            out_copy(blk1 - 2, ob1, so1).wait()

        compute(xb1, ob1)
        out_copy(blk1, ob1, so1).start()

        @pl.when(g < NBP - 1)
        def _():
            in_copy(blk1 + 2, xb1, si1).start()

    out_copy(NB - 2, ob0, so0).wait()
    out_copy(NB - 1, ob1, so1).wait()


def kernel(x, w, b):
    mesh = plsc.VectorSubcoreMesh(core_axis_name="c", subcore_axis_name="s")
    run = functools.partial(
        pl.kernel,
        out_type=jax.ShapeDtypeStruct((B, 1, N), jnp.float32),
        mesh=mesh,
        scratch_types=[
            pltpu.VMEM((RB, W), jnp.float32),
            pltpu.VMEM((RB, W), jnp.float32),
            pltpu.VMEM((RB, W), jnp.float32),
            pltpu.VMEM((RB, W), jnp.float32),
            pltpu.VMEM((W2,), jnp.float32),
            pltpu.VMEM((W2,), jnp.float32),
            pltpu.SemaphoreType.DMA,
            pltpu.SemaphoreType.DMA,
            pltpu.SemaphoreType.DMA,
            pltpu.SemaphoreType.DMA,
        ],
        compiler_params=pltpu.CompilerParams(needs_layout_passes=False),
    )(_body)
    return run(x, w, b)
